# baseline (speedup 1.0000x reference)
import jax
import jax.numpy as jnp
from jax import lax
from jax.experimental import pallas as pl
from jax.experimental.pallas import tpu as pltpu
from jax.experimental.pallas import tpu_sc as plsc

_NUM_ROWS = 64
_ROW_STRIDE = 700
_D = 256
_NC = 2
_NS = 16
_ROWS_PER_WORKER = _NUM_ROWS // (_NC * _NS)


def _body(table_hbm, out_hbm, sem):
    wid = lax.axis_index("s") * _NC + lax.axis_index("c")
    base = wid * _ROWS_PER_WORKER
    copies = []
    for j in range(_ROWS_PER_WORKER):
        copies.append(
            pltpu.make_async_copy(
                table_hbm.at[pl.ds((base + j) * _ROW_STRIDE, 1)],
                out_hbm.at[pl.ds(base + j, 1)],
                sem,
            )
        )
    for c in copies:
        c.start()
    for c in copies:
        c.wait()


def kernel(inputs):
    gathered = pl.kernel(
        _body,
        out_type=jax.ShapeDtypeStruct((_NUM_ROWS, _D), jnp.float32),
        mesh=plsc.VectorSubcoreMesh(
            core_axis_name="c", subcore_axis_name="s",
            num_cores=_NC, num_subcores=_NS,
        ),
        scratch_types=[pltpu.SemaphoreType.DMA],
    )(inputs)
    return jnp.reshape(gathered, (1, _NUM_ROWS * _D))


# speedup vs baseline: 1.1278x; 1.1278x over previous
import jax
import jax.numpy as jnp
from jax import lax
from jax.experimental import pallas as pl
from jax.experimental.pallas import tpu as pltpu
from jax.experimental.pallas import tpu_sc as plsc

_NUM_ROWS = 64
_ROW_STRIDE = 700
_D = 256
_NC = 2
_ROWS_PER_CORE = _NUM_ROWS // _NC


def _body(table_hbm, out_hbm, buf_spmem, sem):
    cid = lax.axis_index("c")
    base = cid * _ROWS_PER_CORE
    copies = []
    for j in range(_ROWS_PER_CORE):
        copies.append(
            pltpu.make_async_copy(
                table_hbm.at[pl.ds((base + j) * _ROW_STRIDE, 1)],
                buf_spmem.at[pl.ds(j, 1)],
                sem,
            )
        )
    for c in copies:
        c.start()
    for c in copies:
        c.wait()
    pltpu.sync_copy(buf_spmem, out_hbm.at[pl.ds(base, _ROWS_PER_CORE)])


def kernel(inputs):
    gathered = pl.kernel(
        _body,
        out_type=jax.ShapeDtypeStruct((_NUM_ROWS, _D), jnp.float32),
        mesh=plsc.ScalarSubcoreMesh(axis_name="c", num_cores=_NC),
        scratch_types=[
            pltpu.VMEM_SHARED((_ROWS_PER_CORE, _D), jnp.float32),
            pltpu.SemaphoreType.DMA,
        ],
    )(inputs)
    return jnp.reshape(gathered, (1, _NUM_ROWS * _D))
